# baseline (speedup 1.0000x reference)
import functools

import jax
import jax.numpy as jnp
from jax import lax
from jax.experimental import pallas as pl
from jax.experimental.pallas import tpu as pltpu
from jax.experimental.pallas import tpu_sc as plsc

N = 10000
E = 320000
IN_CH = 128
HID = 256
OUT_CH = 128
N_CLASSES = 16
N_GRAPHS = 64
EPS = 1e-5

NC = 2
NS = 16
CH = 128

G = 16
NG = -(-((E + NS * CH - 1) // (NS * CH)) // G)
T_MSG = NG * G
EP_MSG = NS * CH * T_MSG
T_DEG = (E + NC * NS * CH - 1) // (NC * NS * CH)
EP_DEG = NC * NS * CH * T_DEG

ACC_ROWS = 10112
RPT = ACC_ROWS // NS
T_ES = EP_MSG // (NC * NS * CH)
NG_ES = T_ES // G
DEG_ROWS = 10240
DEG_SLICE = DEG_ROWS // NS


def _gather2(hs_hbm, idxs_v, j, buf, semA, semB):
    del semB
    pltpu.async_copy(hs_hbm.at[idxs_v.at[j]], buf, semA)


def _gather2_wait(hs_hbm, idxs_v, j, buf, semA, semB):
    del semB
    pltpu.make_async_copy(hs_hbm.at[idxs_v.at[j]], buf, semA).wait()


def _msg_body(dh, hs_hbm, srcm_hbm, dstm_hbm, out_hbm,
              idxs_v, idxd_v, rows0, rows1, acc_sh,
              sem0a, sem0b, sem1a, sem1b):
    c = lax.axis_index("c")
    s = lax.axis_index("s")
    w = c * NS + s

    pltpu.sync_copy(hs_hbm.at[pl.ds(c * ACC_ROWS + s * RPT, RPT)],
                    acc_sh.at[pl.ds(s * RPT, RPT)])
    plsc.subcore_barrier()

    @pl.loop(0, NG)
    def _(g):
        pltpu.sync_copy(srcm_hbm.at[w, pl.ds(g * G, G)], idxs_v)
        pltpu.sync_copy(dstm_hbm.at[s, pl.ds(g * G, G)], idxd_v)
        _gather2(hs_hbm, idxs_v, 0, rows0, sem0a, sem0b)

        @pl.loop(0, G, step=2)
        def _(j0):
            for b in range(2):
                j = j0 + b
                cur, ca, cb = (rows0, sem0a, sem0b) if b == 0 else \
                              (rows1, sem1a, sem1b)
                nxt, na, nb = (rows1, sem1a, sem1b) if b == 0 else \
                              (rows0, sem0a, sem0b)
                _gather2_wait(hs_hbm, idxs_v, j, cur, ca, cb)
                if b == 0:
                    _gather2(hs_hbm, idxs_v, j + 1, nxt, na, nb)
                else:
                    @pl.when(j0 + 2 < G)
                    def _():
                        _gather2(hs_hbm, idxs_v, j0 + 2, nxt, na, nb)
                pltpu.sync_copy(cur, acc_sh.at[idxd_v.at[j]], add=True)

    plsc.subcore_barrier()
    pltpu.sync_copy(acc_sh.at[pl.ds(s * RPT, RPT)],
                    out_hbm.at[pl.ds(c * ACC_ROWS + s * RPT, RPT)])


@functools.lru_cache(maxsize=None)
def _make_msg_kernel(dh):
    mesh = plsc.VectorSubcoreMesh(core_axis_name="c", subcore_axis_name="s",
                                  num_cores=NC, num_subcores=NS)
    return pl.kernel(
        functools.partial(_msg_body, dh),
        out_type=jax.ShapeDtypeStruct((NC * ACC_ROWS, dh), jnp.float32),
        mesh=mesh,
        scratch_types=[
            pltpu.VMEM((G, CH), jnp.int32),
            pltpu.VMEM((G, CH), jnp.int32),
            pltpu.VMEM((CH, dh), jnp.float32),
            pltpu.VMEM((CH, dh), jnp.float32),
            pltpu.VMEM_SHARED((ACC_ROWS, dh), jnp.float32),
            pltpu.SemaphoreType.DMA,
            pltpu.SemaphoreType.DMA,
            pltpu.SemaphoreType.DMA,
            pltpu.SemaphoreType.DMA,
        ],
    )


def _msg_es_body(hs_hbm, srcm_hbm, dstm_hbm, zeros_hbm, out_hbm,
                 idxs_v, idxd_v, rows0, rows1, acc_sh,
                 sem0a, sem0b, sem1a, sem1b):
    c = lax.axis_index("c")
    s = lax.axis_index("s")
    w = c * NS + s

    @pl.when(c == 0)
    def _():
        pltpu.sync_copy(hs_hbm.at[pl.ds(s * RPT, RPT)],
                        acc_sh.at[pl.ds(s * RPT, RPT)])

    @pl.when(c == 1)
    def _():
        pltpu.sync_copy(zeros_hbm, acc_sh.at[pl.ds(s * RPT, RPT)])

    plsc.subcore_barrier()

    @pl.loop(0, NG_ES)
    def _(g):
        pltpu.sync_copy(srcm_hbm.at[w, pl.ds(g * G, G)], idxs_v)
        pltpu.sync_copy(dstm_hbm.at[w, pl.ds(g * G, G)], idxd_v)
        _gather2(hs_hbm, idxs_v, 0, rows0, sem0a, sem0b)

        @pl.loop(0, G, step=2)
        def _(j0):
            for b in range(2):
                j = j0 + b
                cur, ca, cb = (rows0, sem0a, sem0b) if b == 0 else \
                              (rows1, sem1a, sem1b)
                nxt, na, nb = (rows1, sem1a, sem1b) if b == 0 else \
                              (rows0, sem0a, sem0b)
                _gather2_wait(hs_hbm, idxs_v, j, cur, ca, cb)
                if b == 0:
                    _gather2(hs_hbm, idxs_v, j + 1, nxt, na, nb)
                else:
                    @pl.when(j0 + 2 < G)
                    def _():
                        _gather2(hs_hbm, idxs_v, j0 + 2, nxt, na, nb)
                pltpu.sync_copy(cur, acc_sh.at[idxd_v.at[j]], add=True)

    plsc.subcore_barrier()
    pltpu.sync_copy(acc_sh.at[pl.ds(s * RPT, RPT)],
                    out_hbm.at[pl.ds(c * ACC_ROWS + s * RPT, RPT)])


@functools.lru_cache(maxsize=None)
def _make_msg_es_kernel(dh):
    mesh = plsc.VectorSubcoreMesh(core_axis_name="c", subcore_axis_name="s",
                                  num_cores=NC, num_subcores=NS)
    return pl.kernel(
        _msg_es_body,
        out_type=jax.ShapeDtypeStruct((NC * ACC_ROWS, dh), jnp.float32),
        mesh=mesh,
        scratch_types=[
            pltpu.VMEM((G, CH), jnp.int32),
            pltpu.VMEM((G, CH), jnp.int32),
            pltpu.VMEM((CH, dh), jnp.float32),
            pltpu.VMEM((CH, dh), jnp.float32),
            pltpu.VMEM_SHARED((ACC_ROWS, dh), jnp.float32),
            pltpu.SemaphoreType.DMA,
            pltpu.SemaphoreType.DMA,
            pltpu.SemaphoreType.DMA,
            pltpu.SemaphoreType.DMA,
        ],
    )


def _deg_body(dstd_hbm, out_hbm, idx_v, ones_v, stage_v, acc_sh, sem):
    c = lax.axis_index("c")
    s = lax.axis_index("s")
    w = c * NS + s

    pltpu.sync_copy(dstd_hbm.at[w], idx_v)
    for i in range(CH // 16):
        ones_v[pl.ds(i * 16, 16)] = jnp.full((16,), 1.0, jnp.float32)
    for i in range(DEG_SLICE // 16):
        stage_v[pl.ds(i * 16, 16)] = jnp.zeros((16,), jnp.float32)
    pltpu.sync_copy(stage_v, acc_sh.at[pl.ds(s * DEG_SLICE, DEG_SLICE)])
    plsc.subcore_barrier()

    @pl.loop(0, T_DEG)
    def _(j):
        pltpu.sync_copy(ones_v, acc_sh.at[idx_v.at[j]], add=True)

    plsc.subcore_barrier()
    pltpu.sync_copy(acc_sh.at[pl.ds(s * DEG_SLICE, DEG_SLICE)], stage_v)
    pltpu.sync_copy(stage_v,
                    out_hbm.at[pl.ds(c * DEG_ROWS + s * DEG_SLICE, DEG_SLICE)])


@functools.lru_cache(maxsize=None)
def _make_deg_kernel():
    return pl.kernel(
        _deg_body,
        out_type=jax.ShapeDtypeStruct((NC * DEG_ROWS,), jnp.float32),
        mesh=plsc.VectorSubcoreMesh(core_axis_name="c", subcore_axis_name="s",
                                    num_cores=NC, num_subcores=NS),
        scratch_types=[
            pltpu.VMEM((T_DEG, CH), jnp.int32),
            pltpu.VMEM((CH,), jnp.float32),
            pltpu.VMEM((DEG_SLICE,), jnp.float32),
            pltpu.VMEM_SHARED((DEG_ROWS,), jnp.float32),
            pltpu.SemaphoreType.DMA,
        ],
    )


def _dinv_from(degp_ref):
    deg = degp_ref[:, 0:1] + degp_ref[:, 1:2] + 1.0
    return lax.rsqrt(deg)


def _pre_body(degp_ref, x_ref, w1_ref, out_ref):
    dinv = _dinv_from(degp_ref)
    hs = jnp.dot(x_ref[...], w1_ref[...],
                 preferred_element_type=jnp.float32) * dinv
    out_ref[0, :N] = hs[:, :HID // 2]
    out_ref[1, :N] = hs[:, HID // 2:]


_pre_kernel = pl.pallas_call(
    _pre_body,
    out_shape=jax.ShapeDtypeStruct((2, ACC_ROWS, HID // 2), jnp.float32),
)


def _mid_body(din, dout, split, acc_ref, degp_ref, b_ref, g_ref, be_ref,
              wn_ref, out_ref):
    dinv = _dinv_from(degp_ref)
    h = jnp.concatenate([acc_ref[0, :N], acc_ref[1, :N]], axis=1)
    t = h * dinv + b_ref[...]
    m = jnp.mean(t, axis=0, keepdims=True)
    v = jnp.mean((t - m) * (t - m), axis=0, keepdims=True)
    t = (t - m) * lax.rsqrt(v + EPS) * g_ref[...] + be_ref[...]
    t = jnp.maximum(t, 0.0)
    hs = jnp.dot(t, wn_ref[...], preferred_element_type=jnp.float32) * dinv
    if split:
        out_ref[0, :N] = hs[:, :dout // 2]
        out_ref[1, :N] = hs[:, dout // 2:]
    else:
        out_ref[:N] = hs


def _make_mid_kernel(din, dout, split=True):
    shape = (2, ACC_ROWS, dout // 2) if split else (ACC_ROWS, dout)
    return pl.pallas_call(
        functools.partial(_mid_body, din, dout, split),
        out_shape=jax.ShapeDtypeStruct(shape, jnp.float32),
    )


def _post_body(acc_ref, degp_ref, b_ref, g_ref, be_ref, batch_ref,
               wl_ref, bl_ref, out_ref):
    dinv = _dinv_from(degp_ref)
    h = acc_ref[0, :N] + acc_ref[1, :N]
    t = h * dinv + b_ref[...]
    m = jnp.mean(t, axis=0, keepdims=True)
    v = jnp.mean((t - m) * (t - m), axis=0, keepdims=True)
    t = (t - m) * lax.rsqrt(v + EPS) * g_ref[...] + be_ref[...]
    t = jnp.maximum(t, 0.0)
    gids = lax.broadcasted_iota(jnp.int32, (N_GRAPHS, N), 0)
    onehot = jnp.where(batch_ref[...] == gids, 1.0, 0.0)
    sums = jnp.dot(onehot, t, preferred_element_type=jnp.float32)
    cnt = jnp.sum(onehot, axis=1, keepdims=True)
    pooled = sums / jnp.maximum(cnt, 1.0)
    out_ref[...] = jnp.dot(pooled, wl_ref[...],
                           preferred_element_type=jnp.float32) + bl_ref[...]


_post_kernel = pl.pallas_call(
    _post_body,
    out_shape=jax.ShapeDtypeStruct((N_GRAPHS, N_CLASSES), jnp.float32),
)


def kernel(x, edge_index, batch, W1, b1, g1, be1, W2, b2, g2, be2,
           W3, b3, g3, be3, Wl, bl):
    src = edge_index[0].astype(jnp.int32)
    dst = edge_index[1].astype(jnp.int32)

    perm = jnp.argsort(src)
    srcs = src[perm]
    dsts = dst[perm]

    srcp = jnp.concatenate([srcs, jnp.zeros((EP_MSG - E,), jnp.int32)])
    dstp = jnp.concatenate([dsts, jnp.full((EP_MSG - E,), N, jnp.int32)])
    srcm = (jnp.stack([srcp, srcp + ACC_ROWS])
            .reshape(NC * NS, T_MSG, CH))
    dstm = dstp.reshape(NS, T_MSG, CH)
    srcm3 = srcp.reshape(NC * NS, T_ES, CH)
    dstm3 = dstp.reshape(NC * NS, T_ES, CH)
    dstd = (jnp.concatenate([dst, jnp.full((EP_DEG - E,), N, jnp.int32)])
            .reshape(NC * NS, T_DEG, CH))
    zeros_rpt = jnp.zeros((RPT, HID // 2), jnp.float32)
    degp = _make_deg_kernel()(dstd)
    degp2 = degp.reshape(NC, DEG_ROWS)[:, :N].T

    hs1 = _pre_kernel(degp2, x, W1).reshape(NC * ACC_ROWS, HID // 2)
    acc1 = (_make_msg_kernel(HID // 2)(hs1, srcm, dstm)
            .reshape(NC, ACC_ROWS, HID // 2))
    hs2 = (_make_mid_kernel(HID, HID)(acc1, degp2, b1, g1, be1, W2)
           .reshape(NC * ACC_ROWS, HID // 2))
    acc2 = (_make_msg_kernel(HID // 2)(hs2, srcm, dstm)
            .reshape(NC, ACC_ROWS, HID // 2))
    hs3 = _make_mid_kernel(HID, OUT_CH, split=False)(acc2, degp2, b2, g2,
                                                     be2, W3)
    acc3 = (_make_msg_es_kernel(OUT_CH)(hs3, srcm3, dstm3, zeros_rpt)
            .reshape(NC, ACC_ROWS, OUT_CH))
    return _post_kernel(acc3, degp2, b3, g3, be3,
                        batch.astype(jnp.int32).reshape(1, N), Wl, bl)


# speedup vs baseline: 1.2840x; 1.2840x over previous
import functools

import jax
import jax.numpy as jnp
from jax import lax
from jax.experimental import pallas as pl
from jax.experimental.pallas import tpu as pltpu
from jax.experimental.pallas import tpu_sc as plsc

N = 10000
E = 320000
IN_CH = 128
HID = 256
OUT_CH = 128
N_CLASSES = 16
N_GRAPHS = 64
EPS = 1e-5

NC = 2
NS = 16
CH = 128

G = 16
NG = -(-((E + NS * CH - 1) // (NS * CH)) // G)
T_MSG = NG * G
EP_MSG = NS * CH * T_MSG
T_DEG = (E + NC * NS * CH - 1) // (NC * NS * CH)
EP_DEG = NC * NS * CH * T_DEG

ACC_ROWS = 10112
RPT = ACC_ROWS // NS
T_ES = EP_MSG // (NC * NS * CH)
NG_ES = T_ES // G
DEG_ROWS = 10240
DEG_SLICE = DEG_ROWS // NS


def _gather2(hs_hbm, idxs_v, j, buf, semA, semB):
    del semB
    pltpu.async_copy(hs_hbm.at[idxs_v.at[j]], buf, semA)


def _gather2_wait(hs_hbm, idxs_v, j, buf, semA, semB):
    del semB
    pltpu.make_async_copy(hs_hbm.at[idxs_v.at[j]], buf, semA).wait()


def _msg_body(dh, hs_hbm, srcm_hbm, dstm_hbm, out_hbm,
              idxs_v, idxd_v, rows0, rows1, acc_sh,
              sem0a, sem0b, sem1a, sem1b):
    c = lax.axis_index("c")
    s = lax.axis_index("s")
    w = c * NS + s

    pltpu.sync_copy(hs_hbm.at[pl.ds(c * ACC_ROWS + s * RPT, RPT)],
                    acc_sh.at[pl.ds(s * RPT, RPT)])
    plsc.subcore_barrier()

    @pl.loop(0, NG)
    def _(g):
        pltpu.sync_copy(srcm_hbm.at[w, pl.ds(g * G, G)], idxs_v)
        pltpu.sync_copy(dstm_hbm.at[s, pl.ds(g * G, G)], idxd_v)
        _gather2(hs_hbm, idxs_v, 0, rows0, sem0a, sem0b)

        @pl.loop(0, G, step=2)
        def _(j0):
            for b in range(2):
                j = j0 + b
                cur, ca, cb = (rows0, sem0a, sem0b) if b == 0 else \
                              (rows1, sem1a, sem1b)
                nxt, na, nb = (rows1, sem1a, sem1b) if b == 0 else \
                              (rows0, sem0a, sem0b)
                _gather2_wait(hs_hbm, idxs_v, j, cur, ca, cb)
                if b == 0:
                    _gather2(hs_hbm, idxs_v, j + 1, nxt, na, nb)
                else:
                    @pl.when(j0 + 2 < G)
                    def _():
                        _gather2(hs_hbm, idxs_v, j0 + 2, nxt, na, nb)
                pltpu.sync_copy(cur, acc_sh.at[idxd_v.at[j]], add=True)

    plsc.subcore_barrier()
    pltpu.sync_copy(acc_sh.at[pl.ds(s * RPT, RPT)],
                    out_hbm.at[pl.ds(c * ACC_ROWS + s * RPT, RPT)])


@functools.lru_cache(maxsize=None)
def _make_msg_kernel(dh):
    mesh = plsc.VectorSubcoreMesh(core_axis_name="c", subcore_axis_name="s",
                                  num_cores=NC, num_subcores=NS)
    return pl.kernel(
        functools.partial(_msg_body, dh),
        out_type=jax.ShapeDtypeStruct((NC * ACC_ROWS, dh), jnp.float32),
        mesh=mesh,
        scratch_types=[
            pltpu.VMEM((G, CH), jnp.int32),
            pltpu.VMEM((G, CH), jnp.int32),
            pltpu.VMEM((CH, dh), jnp.float32),
            pltpu.VMEM((CH, dh), jnp.float32),
            pltpu.VMEM_SHARED((ACC_ROWS, dh), jnp.float32),
            pltpu.SemaphoreType.DMA,
            pltpu.SemaphoreType.DMA,
            pltpu.SemaphoreType.DMA,
            pltpu.SemaphoreType.DMA,
        ],
    )


def _msg_es_body(hs_hbm, srcm_hbm, dstm_hbm, zeros_hbm, out_hbm,
                 idxs_v, idxd_v, rows0, rows1, acc_sh,
                 sem0a, sem0b, sem1a, sem1b):
    c = lax.axis_index("c")
    s = lax.axis_index("s")
    w = c * NS + s

    @pl.when(c == 0)
    def _():
        pltpu.sync_copy(hs_hbm.at[pl.ds(s * RPT, RPT)],
                        acc_sh.at[pl.ds(s * RPT, RPT)])

    @pl.when(c == 1)
    def _():
        pltpu.sync_copy(zeros_hbm, acc_sh.at[pl.ds(s * RPT, RPT)])

    plsc.subcore_barrier()

    @pl.loop(0, NG_ES)
    def _(g):
        pltpu.sync_copy(srcm_hbm.at[w, pl.ds(g * G, G)], idxs_v)
        pltpu.sync_copy(dstm_hbm.at[w, pl.ds(g * G, G)], idxd_v)
        _gather2(hs_hbm, idxs_v, 0, rows0, sem0a, sem0b)

        @pl.loop(0, G, step=2)
        def _(j0):
            for b in range(2):
                j = j0 + b
                cur, ca, cb = (rows0, sem0a, sem0b) if b == 0 else \
                              (rows1, sem1a, sem1b)
                nxt, na, nb = (rows1, sem1a, sem1b) if b == 0 else \
                              (rows0, sem0a, sem0b)
                _gather2_wait(hs_hbm, idxs_v, j, cur, ca, cb)
                if b == 0:
                    _gather2(hs_hbm, idxs_v, j + 1, nxt, na, nb)
                else:
                    @pl.when(j0 + 2 < G)
                    def _():
                        _gather2(hs_hbm, idxs_v, j0 + 2, nxt, na, nb)
                pltpu.sync_copy(cur, acc_sh.at[idxd_v.at[j]], add=True)

    plsc.subcore_barrier()
    pltpu.sync_copy(acc_sh.at[pl.ds(s * RPT, RPT)],
                    out_hbm.at[pl.ds(c * ACC_ROWS + s * RPT, RPT)])


@functools.lru_cache(maxsize=None)
def _make_msg_es_kernel(dh):
    mesh = plsc.VectorSubcoreMesh(core_axis_name="c", subcore_axis_name="s",
                                  num_cores=NC, num_subcores=NS)
    return pl.kernel(
        _msg_es_body,
        out_type=jax.ShapeDtypeStruct((NC * ACC_ROWS, dh), jnp.float32),
        mesh=mesh,
        scratch_types=[
            pltpu.VMEM((G, CH), jnp.int32),
            pltpu.VMEM((G, CH), jnp.int32),
            pltpu.VMEM((CH, dh), jnp.float32),
            pltpu.VMEM((CH, dh), jnp.float32),
            pltpu.VMEM_SHARED((ACC_ROWS, dh), jnp.float32),
            pltpu.SemaphoreType.DMA,
            pltpu.SemaphoreType.DMA,
            pltpu.SemaphoreType.DMA,
            pltpu.SemaphoreType.DMA,
        ],
    )


def _deg_body(dstd_hbm, out_hbm, idx_v, ones_v, stage_v, acc_sh, sem):
    c = lax.axis_index("c")
    s = lax.axis_index("s")
    w = c * NS + s

    pltpu.sync_copy(dstd_hbm.at[w], idx_v)
    for i in range(CH // 16):
        ones_v[pl.ds(i * 16, 16)] = jnp.full((16,), 1.0, jnp.float32)
    for i in range(DEG_SLICE // 16):
        stage_v[pl.ds(i * 16, 16)] = jnp.zeros((16,), jnp.float32)
    pltpu.sync_copy(stage_v, acc_sh.at[pl.ds(s * DEG_SLICE, DEG_SLICE)])
    plsc.subcore_barrier()

    @pl.loop(0, T_DEG)
    def _(j):
        pltpu.sync_copy(ones_v, acc_sh.at[idx_v.at[j]], add=True)

    plsc.subcore_barrier()
    pltpu.sync_copy(acc_sh.at[pl.ds(s * DEG_SLICE, DEG_SLICE)], stage_v)
    pltpu.sync_copy(stage_v,
                    out_hbm.at[pl.ds(c * DEG_ROWS + s * DEG_SLICE, DEG_SLICE)])


@functools.lru_cache(maxsize=None)
def _make_deg_kernel():
    return pl.kernel(
        _deg_body,
        out_type=jax.ShapeDtypeStruct((NC * DEG_ROWS,), jnp.float32),
        mesh=plsc.VectorSubcoreMesh(core_axis_name="c", subcore_axis_name="s",
                                    num_cores=NC, num_subcores=NS),
        scratch_types=[
            pltpu.VMEM((T_DEG, CH), jnp.int32),
            pltpu.VMEM((CH,), jnp.float32),
            pltpu.VMEM((DEG_SLICE,), jnp.float32),
            pltpu.VMEM_SHARED((DEG_ROWS,), jnp.float32),
            pltpu.SemaphoreType.DMA,
        ],
    )


def _dinv_from(degp_ref):
    deg = degp_ref[:, 0:1] + degp_ref[:, 1:2] + 1.0
    return lax.rsqrt(deg)


def _pre_body(degp_ref, x_ref, out_ref):
    dinv = _dinv_from(degp_ref)
    xs = x_ref[...] * dinv
    out_ref[0, :N] = xs
    out_ref[1, :N] = xs


_pre_kernel = pl.pallas_call(
    _pre_body,
    out_shape=jax.ShapeDtypeStruct((2, ACC_ROWS, IN_CH), jnp.float32),
)


def _l1_body(acc_ref, degp_ref, w1_ref, b_ref, g_ref, be_ref, wn_ref,
             out_ref):
    dinv = _dinv_from(degp_ref)
    h = acc_ref[0, :N] + acc_ref[1, :N]
    t = jnp.dot(h * dinv, w1_ref[...],
                preferred_element_type=jnp.float32) + b_ref[...]
    m = jnp.mean(t, axis=0, keepdims=True)
    v = jnp.mean((t - m) * (t - m), axis=0, keepdims=True)
    t = (t - m) * lax.rsqrt(v + EPS) * g_ref[...] + be_ref[...]
    t = jnp.maximum(t, 0.0)
    hs = jnp.dot(t, wn_ref[...], preferred_element_type=jnp.float32) * dinv
    out_ref[0, :N] = hs[:, :HID // 2]
    out_ref[1, :N] = hs[:, HID // 2:]


_l1_kernel = pl.pallas_call(
    _l1_body,
    out_shape=jax.ShapeDtypeStruct((2, ACC_ROWS, HID // 2), jnp.float32),
)


def _mid_body(din, dout, split, acc_ref, degp_ref, b_ref, g_ref, be_ref,
              wn_ref, out_ref):
    dinv = _dinv_from(degp_ref)
    h = jnp.concatenate([acc_ref[0, :N], acc_ref[1, :N]], axis=1)
    t = h * dinv + b_ref[...]
    m = jnp.mean(t, axis=0, keepdims=True)
    v = jnp.mean((t - m) * (t - m), axis=0, keepdims=True)
    t = (t - m) * lax.rsqrt(v + EPS) * g_ref[...] + be_ref[...]
    t = jnp.maximum(t, 0.0)
    hs = jnp.dot(t, wn_ref[...], preferred_element_type=jnp.float32) * dinv
    if split:
        out_ref[0, :N] = hs[:, :dout // 2]
        out_ref[1, :N] = hs[:, dout // 2:]
    else:
        out_ref[0, :N] = hs
        out_ref[1, :N] = hs


def _make_mid_kernel(din, dout, split=True):
    shape = (2, ACC_ROWS, dout // 2) if split else (2, ACC_ROWS, dout)
    return pl.pallas_call(
        functools.partial(_mid_body, din, dout, split),
        out_shape=jax.ShapeDtypeStruct(shape, jnp.float32),
    )


def _post_body(acc_ref, degp_ref, b_ref, g_ref, be_ref, batch_ref,
               wl_ref, bl_ref, out_ref):
    dinv = _dinv_from(degp_ref)
    h = acc_ref[0, :N] + acc_ref[1, :N]
    t = h * dinv + b_ref[...]
    m = jnp.mean(t, axis=0, keepdims=True)
    v = jnp.mean((t - m) * (t - m), axis=0, keepdims=True)
    t = (t - m) * lax.rsqrt(v + EPS) * g_ref[...] + be_ref[...]
    t = jnp.maximum(t, 0.0)
    gids = lax.broadcasted_iota(jnp.int32, (N_GRAPHS, N), 0)
    onehot = jnp.where(batch_ref[...] == gids, 1.0, 0.0)
    sums = jnp.dot(onehot, t, preferred_element_type=jnp.float32)
    cnt = jnp.sum(onehot, axis=1, keepdims=True)
    pooled = sums / jnp.maximum(cnt, 1.0)
    out_ref[...] = jnp.dot(pooled, wl_ref[...],
                           preferred_element_type=jnp.float32) + bl_ref[...]


_post_kernel = pl.pallas_call(
    _post_body,
    out_shape=jax.ShapeDtypeStruct((N_GRAPHS, N_CLASSES), jnp.float32),
)


def kernel(x, edge_index, batch, W1, b1, g1, be1, W2, b2, g2, be2,
           W3, b3, g3, be3, Wl, bl):
    src = edge_index[0].astype(jnp.int32)
    dst = edge_index[1].astype(jnp.int32)

    srcp = jnp.concatenate([src, jnp.zeros((EP_MSG - E,), jnp.int32)])
    dstp = jnp.concatenate([dst, jnp.full((EP_MSG - E,), N, jnp.int32)])
    srcm = (jnp.stack([srcp, srcp + ACC_ROWS])
            .reshape(NC * NS, T_MSG, CH))
    dstm = dstp.reshape(NS, T_MSG, CH)
    core_off = (jnp.arange(NC * NS, dtype=jnp.int32)[:, None, None]
                // NS) * ACC_ROWS
    srcm3 = srcp.reshape(NC * NS, T_ES, CH) + core_off
    dstm3 = dstp.reshape(NC * NS, T_ES, CH)
    dstd = (jnp.concatenate([dst, jnp.full((EP_DEG - E,), N, jnp.int32)])
            .reshape(NC * NS, T_DEG, CH))
    zeros_rpt = jnp.zeros((RPT, HID // 2), jnp.float32)
    degp = _make_deg_kernel()(dstd)
    degp2 = degp.reshape(NC, DEG_ROWS)[:, :N].T

    xs = _pre_kernel(degp2, x).reshape(NC * ACC_ROWS, IN_CH)
    accx = (_make_msg_es_kernel(IN_CH)(xs, srcm3, dstm3, zeros_rpt)
            .reshape(NC, ACC_ROWS, IN_CH))
    hs2 = (_l1_kernel(accx, degp2, W1, b1, g1, be1, W2)
           .reshape(NC * ACC_ROWS, HID // 2))
    acc2 = (_make_msg_kernel(HID // 2)(hs2, srcm, dstm)
            .reshape(NC, ACC_ROWS, HID // 2))
    hs3 = (_make_mid_kernel(HID, OUT_CH, split=False)(acc2, degp2, b2, g2,
                                                      be2, W3)
           .reshape(NC * ACC_ROWS, OUT_CH))
    acc3 = (_make_msg_es_kernel(OUT_CH)(hs3, srcm3, dstm3, zeros_rpt)
            .reshape(NC, ACC_ROWS, OUT_CH))
    return _post_kernel(acc3, degp2, b3, g3, be3,
                        batch.astype(jnp.int32).reshape(1, N), Wl, bl)


# speedup vs baseline: 1.4065x; 1.0954x over previous
import functools

import jax
import jax.numpy as jnp
from jax import lax
from jax.experimental import pallas as pl
from jax.experimental.pallas import tpu as pltpu
from jax.experimental.pallas import tpu_sc as plsc

N = 10000
E = 320000
IN_CH = 128
HID = 256
OUT_CH = 128
N_CLASSES = 16
N_GRAPHS = 64
EPS = 1e-5

NC = 2
NS = 16
CH = 128

G = 16
NG = -(-((E + NS * CH - 1) // (NS * CH)) // G)
T_MSG = NG * G
EP_MSG = NS * CH * T_MSG
T_DEG = (E + NC * NS * CH - 1) // (NC * NS * CH)
EP_DEG = NC * NS * CH * T_DEG

ACC_ROWS = 10112
RPT = ACC_ROWS // NS
T_ES = EP_MSG // (NC * NS * CH)
NG_ES = T_ES // G
DEG_ROWS = 10240
DEG_SLICE = DEG_ROWS // NS


def _gather2(hs_hbm, idxs_v, j, buf, semA, semB):
    del semB
    pltpu.async_copy(hs_hbm.at[idxs_v.at[j]], buf, semA)


def _gather2_wait(hs_hbm, idxs_v, j, buf, semA, semB):
    del semB
    pltpu.make_async_copy(hs_hbm.at[idxs_v.at[j]], buf, semA).wait()


def _msg_body(dh, hs_hbm, srcm_hbm, dstm_hbm, zeros_hbm, out_hbm,
              idxs_v, idxd_v, rows0, rows1, acc_sh,
              sem0a, sem0b, sem1a, sem1b):
    c = lax.axis_index("c")
    s = lax.axis_index("s")
    w = c * NS + s

    pltpu.sync_copy(zeros_hbm, acc_sh.at[pl.ds(s * RPT, RPT)])
    plsc.subcore_barrier()

    @pl.loop(0, NG)
    def _(g):
        pltpu.sync_copy(srcm_hbm.at[w, pl.ds(g * G, G)], idxs_v)
        pltpu.sync_copy(dstm_hbm.at[s, pl.ds(g * G, G)], idxd_v)
        _gather2(hs_hbm, idxs_v, 0, rows0, sem0a, sem0b)

        @pl.loop(0, G, step=2)
        def _(j0):
            for b in range(2):
                j = j0 + b
                cur, ca, cb = (rows0, sem0a, sem0b) if b == 0 else \
                              (rows1, sem1a, sem1b)
                nxt, na, nb = (rows1, sem1a, sem1b) if b == 0 else \
                              (rows0, sem0a, sem0b)
                _gather2_wait(hs_hbm, idxs_v, j, cur, ca, cb)
                if b == 0:
                    _gather2(hs_hbm, idxs_v, j + 1, nxt, na, nb)
                else:
                    @pl.when(j0 + 2 < G)
                    def _():
                        _gather2(hs_hbm, idxs_v, j0 + 2, nxt, na, nb)
                pltpu.sync_copy(cur, acc_sh.at[idxd_v.at[j]], add=True)

    plsc.subcore_barrier()
    pltpu.sync_copy(acc_sh.at[pl.ds(s * RPT, RPT)],
                    out_hbm.at[pl.ds(c * ACC_ROWS + s * RPT, RPT)])


@functools.lru_cache(maxsize=None)
def _make_msg_kernel(dh):
    mesh = plsc.VectorSubcoreMesh(core_axis_name="c", subcore_axis_name="s",
                                  num_cores=NC, num_subcores=NS)
    return pl.kernel(
        functools.partial(_msg_body, dh),
        out_type=jax.ShapeDtypeStruct((NC * ACC_ROWS, dh), jnp.float32),
        mesh=mesh,
        scratch_types=[
            pltpu.VMEM((G, CH), jnp.int32),
            pltpu.VMEM((G, CH), jnp.int32),
            pltpu.VMEM((CH, dh), jnp.float32),
            pltpu.VMEM((CH, dh), jnp.float32),
            pltpu.VMEM_SHARED((ACC_ROWS, dh), jnp.float32),
            pltpu.SemaphoreType.DMA,
            pltpu.SemaphoreType.DMA,
            pltpu.SemaphoreType.DMA,
            pltpu.SemaphoreType.DMA,
        ],
    )


def _msg_es_body(hs_hbm, srcm_hbm, dstm_hbm, zeros_hbm, out_hbm,
                 idxs_v, idxd_v, rows0, rows1, acc_sh,
                 sem0a, sem0b, sem1a, sem1b):
    c = lax.axis_index("c")
    s = lax.axis_index("s")
    w = c * NS + s

    pltpu.sync_copy(zeros_hbm, acc_sh.at[pl.ds(s * RPT, RPT)])
    plsc.subcore_barrier()

    @pl.loop(0, NG_ES)
    def _(g):
        pltpu.sync_copy(srcm_hbm.at[w, pl.ds(g * G, G)], idxs_v)
        pltpu.sync_copy(dstm_hbm.at[w, pl.ds(g * G, G)], idxd_v)
        _gather2(hs_hbm, idxs_v, 0, rows0, sem0a, sem0b)

        @pl.loop(0, G, step=2)
        def _(j0):
            for b in range(2):
                j = j0 + b
                cur, ca, cb = (rows0, sem0a, sem0b) if b == 0 else \
                              (rows1, sem1a, sem1b)
                nxt, na, nb = (rows1, sem1a, sem1b) if b == 0 else \
                              (rows0, sem0a, sem0b)
                _gather2_wait(hs_hbm, idxs_v, j, cur, ca, cb)
                if b == 0:
                    _gather2(hs_hbm, idxs_v, j + 1, nxt, na, nb)
                else:
                    @pl.when(j0 + 2 < G)
                    def _():
                        _gather2(hs_hbm, idxs_v, j0 + 2, nxt, na, nb)
                pltpu.sync_copy(cur, acc_sh.at[idxd_v.at[j]], add=True)

    plsc.subcore_barrier()
    pltpu.sync_copy(acc_sh.at[pl.ds(s * RPT, RPT)],
                    out_hbm.at[pl.ds(c * ACC_ROWS + s * RPT, RPT)])


@functools.lru_cache(maxsize=None)
def _make_msg_es_kernel(dh):
    mesh = plsc.VectorSubcoreMesh(core_axis_name="c", subcore_axis_name="s",
                                  num_cores=NC, num_subcores=NS)
    return pl.kernel(
        _msg_es_body,
        out_type=jax.ShapeDtypeStruct((NC * ACC_ROWS, dh), jnp.float32),
        mesh=mesh,
        scratch_types=[
            pltpu.VMEM((G, CH), jnp.int32),
            pltpu.VMEM((G, CH), jnp.int32),
            pltpu.VMEM((CH, dh), jnp.float32),
            pltpu.VMEM((CH, dh), jnp.float32),
            pltpu.VMEM_SHARED((ACC_ROWS, dh), jnp.float32),
            pltpu.SemaphoreType.DMA,
            pltpu.SemaphoreType.DMA,
            pltpu.SemaphoreType.DMA,
            pltpu.SemaphoreType.DMA,
        ],
    )


def _deg_body(dstd_hbm, out_hbm, idx_v, ones_v, stage_v, acc_sh, sem):
    c = lax.axis_index("c")
    s = lax.axis_index("s")
    w = c * NS + s

    pltpu.sync_copy(dstd_hbm.at[w], idx_v)
    for i in range(CH // 16):
        ones_v[pl.ds(i * 16, 16)] = jnp.full((16,), 1.0, jnp.float32)
    for i in range(DEG_SLICE // 16):
        stage_v[pl.ds(i * 16, 16)] = jnp.zeros((16,), jnp.float32)
    pltpu.sync_copy(stage_v, acc_sh.at[pl.ds(s * DEG_SLICE, DEG_SLICE)])
    plsc.subcore_barrier()

    @pl.loop(0, T_DEG)
    def _(j):
        pltpu.sync_copy(ones_v, acc_sh.at[idx_v.at[j]], add=True)

    plsc.subcore_barrier()
    pltpu.sync_copy(acc_sh.at[pl.ds(s * DEG_SLICE, DEG_SLICE)], stage_v)
    pltpu.sync_copy(stage_v,
                    out_hbm.at[pl.ds(c * DEG_ROWS + s * DEG_SLICE, DEG_SLICE)])


@functools.lru_cache(maxsize=None)
def _make_deg_kernel():
    return pl.kernel(
        _deg_body,
        out_type=jax.ShapeDtypeStruct((NC * DEG_ROWS,), jnp.float32),
        mesh=plsc.VectorSubcoreMesh(core_axis_name="c", subcore_axis_name="s",
                                    num_cores=NC, num_subcores=NS),
        scratch_types=[
            pltpu.VMEM((T_DEG, CH), jnp.int32),
            pltpu.VMEM((CH,), jnp.float32),
            pltpu.VMEM((DEG_SLICE,), jnp.float32),
            pltpu.VMEM_SHARED((DEG_ROWS,), jnp.float32),
            pltpu.SemaphoreType.DMA,
        ],
    )


def _dinv_from(degp_ref):
    deg = degp_ref[:, 0:1] + degp_ref[:, 1:2] + 1.0
    return lax.rsqrt(deg)


def _pre_body(degp_ref, x_ref, out_ref):
    dinv = _dinv_from(degp_ref)
    xs = x_ref[...] * dinv
    out_ref[0, :N] = xs
    out_ref[1, :N] = xs


_pre_kernel = pl.pallas_call(
    _pre_body,
    out_shape=jax.ShapeDtypeStruct((2, ACC_ROWS, IN_CH), jnp.float32),
)


def _l1_body(acc_ref, degp_ref, x_ref, w1_ref, b_ref, g_ref, be_ref, wn_ref,
             out_ref):
    dinv = _dinv_from(degp_ref)
    h = acc_ref[0, :N] + acc_ref[1, :N] + x_ref[...] * dinv
    t = jnp.dot(h * dinv, w1_ref[...],
                preferred_element_type=jnp.float32) + b_ref[...]
    m = jnp.mean(t, axis=0, keepdims=True)
    v = jnp.mean((t - m) * (t - m), axis=0, keepdims=True)
    t = (t - m) * lax.rsqrt(v + EPS) * g_ref[...] + be_ref[...]
    t = jnp.maximum(t, 0.0)
    hs = jnp.dot(t, wn_ref[...], preferred_element_type=jnp.float32) * dinv
    out_ref[0, :N] = hs[:, :HID // 2]
    out_ref[1, :N] = hs[:, HID // 2:]


_l1_kernel = pl.pallas_call(
    _l1_body,
    out_shape=jax.ShapeDtypeStruct((2, ACC_ROWS, HID // 2), jnp.float32),
)


def _mid_body(din, dout, split, acc_ref, degp_ref, hs_ref, b_ref, g_ref,
              be_ref, wn_ref, out_ref):
    dinv = _dinv_from(degp_ref)
    h = jnp.concatenate([acc_ref[0, :N] + hs_ref[0, :N],
                         acc_ref[1, :N] + hs_ref[1, :N]], axis=1)
    t = h * dinv + b_ref[...]
    m = jnp.mean(t, axis=0, keepdims=True)
    v = jnp.mean((t - m) * (t - m), axis=0, keepdims=True)
    t = (t - m) * lax.rsqrt(v + EPS) * g_ref[...] + be_ref[...]
    t = jnp.maximum(t, 0.0)
    hs = jnp.dot(t, wn_ref[...], preferred_element_type=jnp.float32) * dinv
    if split:
        out_ref[0, :N] = hs[:, :dout // 2]
        out_ref[1, :N] = hs[:, dout // 2:]
    else:
        out_ref[0, :N] = hs
        out_ref[1, :N] = hs


def _make_mid_kernel(din, dout, split=True):
    shape = (2, ACC_ROWS, dout // 2) if split else (2, ACC_ROWS, dout)
    return pl.pallas_call(
        functools.partial(_mid_body, din, dout, split),
        out_shape=jax.ShapeDtypeStruct(shape, jnp.float32),
    )


def _post_body(acc_ref, degp_ref, hs_ref, b_ref, g_ref, be_ref, batch_ref,
               wl_ref, bl_ref, out_ref):
    dinv = _dinv_from(degp_ref)
    h = acc_ref[0, :N] + acc_ref[1, :N] + hs_ref[0, :N]
    t = h * dinv + b_ref[...]
    m = jnp.mean(t, axis=0, keepdims=True)
    v = jnp.mean((t - m) * (t - m), axis=0, keepdims=True)
    t = (t - m) * lax.rsqrt(v + EPS) * g_ref[...] + be_ref[...]
    t = jnp.maximum(t, 0.0)
    gids = lax.broadcasted_iota(jnp.int32, (N_GRAPHS, N), 0)
    onehot = jnp.where(batch_ref[...] == gids, 1.0, 0.0)
    sums = jnp.dot(onehot, t, preferred_element_type=jnp.float32)
    cnt = jnp.sum(onehot, axis=1, keepdims=True)
    pooled = sums / jnp.maximum(cnt, 1.0)
    out_ref[...] = jnp.dot(pooled, wl_ref[...],
                           preferred_element_type=jnp.float32) + bl_ref[...]


_post_kernel = pl.pallas_call(
    _post_body,
    out_shape=jax.ShapeDtypeStruct((N_GRAPHS, N_CLASSES), jnp.float32),
)


def kernel(x, edge_index, batch, W1, b1, g1, be1, W2, b2, g2, be2,
           W3, b3, g3, be3, Wl, bl):
    src = edge_index[0].astype(jnp.int32)
    dst = edge_index[1].astype(jnp.int32)

    srcp = jnp.concatenate([src, jnp.zeros((EP_MSG - E,), jnp.int32)])
    dstp = jnp.concatenate([dst, jnp.full((EP_MSG - E,), N, jnp.int32)])
    srcm = (jnp.stack([srcp, srcp + ACC_ROWS])
            .reshape(NC * NS, T_MSG, CH))
    dstm = dstp.reshape(NS, T_MSG, CH)
    core_off = (jnp.arange(NC * NS, dtype=jnp.int32)[:, None, None]
                // NS) * ACC_ROWS
    srcm3 = srcp.reshape(NC * NS, T_ES, CH) + core_off
    dstm3 = dstp.reshape(NC * NS, T_ES, CH)
    dstd = (jnp.concatenate([dst, jnp.full((EP_DEG - E,), N, jnp.int32)])
            .reshape(NC * NS, T_DEG, CH))
    zeros_rpt = jnp.zeros((RPT, HID // 2), jnp.float32)
    degp = _make_deg_kernel()(dstd)
    degp2 = degp.reshape(NC, DEG_ROWS)[:, :N].T

    xs = _pre_kernel(degp2, x).reshape(NC * ACC_ROWS, IN_CH)
    accx = (_make_msg_es_kernel(IN_CH)(xs, srcm3, dstm3, zeros_rpt)
            .reshape(NC, ACC_ROWS, IN_CH))
    hs2 = _l1_kernel(accx, degp2, x, W1, b1, g1, be1, W2)
    acc2 = (_make_msg_kernel(HID // 2)(hs2.reshape(NC * ACC_ROWS, HID // 2),
                                       srcm, dstm, zeros_rpt)
            .reshape(NC, ACC_ROWS, HID // 2))
    hs3 = _make_mid_kernel(HID, OUT_CH, split=False)(acc2, degp2, hs2,
                                                     b2, g2, be2, W3)
    acc3 = (_make_msg_es_kernel(OUT_CH)(hs3.reshape(NC * ACC_ROWS, OUT_CH),
                                        srcm3, dstm3, zeros_rpt)
            .reshape(NC, ACC_ROWS, OUT_CH))
    return _post_kernel(acc3, degp2, hs3, b3, g3, be3,
                        batch.astype(jnp.int32).reshape(1, N), Wl, bl)


# speedup vs baseline: 1.5080x; 1.0722x over previous
import functools

import jax
import jax.numpy as jnp
from jax import lax
from jax.experimental import pallas as pl
from jax.experimental.pallas import tpu as pltpu
from jax.experimental.pallas import tpu_sc as plsc

N = 10000
E = 320000
IN_CH = 128
HID = 256
OUT_CH = 128
N_CLASSES = 16
N_GRAPHS = 64
EPS = 1e-5

NC = 2
NS = 16
CH = 128

G = 16
NG = -(-((E + NS * CH - 1) // (NS * CH)) // G)
T_MSG = NG * G
EP_MSG = NS * CH * T_MSG
T_DEG = (E + NC * NS * CH - 1) // (NC * NS * CH)
EP_DEG = NC * NS * CH * T_DEG

ACC_ROWS = 10112
RPT = ACC_ROWS // NS
GE = 8
CPA = 120
CPB = 40
NGA = CPA // GE
NGB = CPB // GE
DEG_ROWS = 10240
DEG_SLICE = DEG_ROWS // NS


def _gather2(hs_hbm, idxs_v, j, buf, semA, semB):
    del semB
    pltpu.async_copy(hs_hbm.at[idxs_v.at[j]], buf, semA)


def _gather2_wait(hs_hbm, idxs_v, j, buf, semA, semB):
    del semB
    pltpu.make_async_copy(hs_hbm.at[idxs_v.at[j]], buf, semA).wait()


def _msg_body(dh, hs_hbm, srcm_hbm, dstm_hbm, zeros_hbm, out_hbm,
              idxs_v, idxd_v, rows0, rows1, acc_sh,
              sem0a, sem0b, sem1a, sem1b):
    c = lax.axis_index("c")
    s = lax.axis_index("s")
    w = c * NS + s

    pltpu.sync_copy(zeros_hbm, acc_sh.at[pl.ds(s * RPT, RPT)])
    plsc.subcore_barrier()

    @pl.loop(0, NG)
    def _(g):
        pltpu.sync_copy(srcm_hbm.at[w, pl.ds(g * G, G)], idxs_v)
        pltpu.sync_copy(dstm_hbm.at[s, pl.ds(g * G, G)], idxd_v)
        _gather2(hs_hbm, idxs_v, 0, rows0, sem0a, sem0b)

        @pl.loop(0, G, step=2)
        def _(j0):
            for b in range(2):
                j = j0 + b
                cur, ca, cb = (rows0, sem0a, sem0b) if b == 0 else \
                              (rows1, sem1a, sem1b)
                nxt, na, nb = (rows1, sem1a, sem1b) if b == 0 else \
                              (rows0, sem0a, sem0b)
                _gather2_wait(hs_hbm, idxs_v, j, cur, ca, cb)
                if b == 0:
                    _gather2(hs_hbm, idxs_v, j + 1, nxt, na, nb)
                else:
                    @pl.when(j0 + 2 < G)
                    def _():
                        _gather2(hs_hbm, idxs_v, j0 + 2, nxt, na, nb)
                pltpu.sync_copy(cur, acc_sh.at[idxd_v.at[j]], add=True)

    plsc.subcore_barrier()
    pltpu.sync_copy(acc_sh.at[pl.ds(s * RPT, RPT)],
                    out_hbm.at[pl.ds(c * ACC_ROWS + s * RPT, RPT)])


@functools.lru_cache(maxsize=None)
def _make_msg_kernel(dh):
    mesh = plsc.VectorSubcoreMesh(core_axis_name="c", subcore_axis_name="s",
                                  num_cores=NC, num_subcores=NS)
    return pl.kernel(
        functools.partial(_msg_body, dh),
        out_type=jax.ShapeDtypeStruct((NC * ACC_ROWS, dh), jnp.float32),
        mesh=mesh,
        scratch_types=[
            pltpu.VMEM((G, CH), jnp.int32),
            pltpu.VMEM((G, CH), jnp.int32),
            pltpu.VMEM((CH, dh), jnp.float32),
            pltpu.VMEM((CH, dh), jnp.float32),
            pltpu.VMEM_SHARED((ACC_ROWS, dh), jnp.float32),
            pltpu.SemaphoreType.DMA,
            pltpu.SemaphoreType.DMA,
            pltpu.SemaphoreType.DMA,
            pltpu.SemaphoreType.DMA,
        ],
    )


def _msg_es_body(hs_hbm, srcA_hbm, dstA_hbm, srcB_hbm, dstB_hbm, zeros_hbm,
                 out_hbm, idxs_v, idxd_v, rows0, rows1, acc_sh,
                 sem0a, sem0b, sem1a, sem1b):
    c = lax.axis_index("c")
    s = lax.axis_index("s")

    pltpu.sync_copy(zeros_hbm, acc_sh.at[pl.ds(s * RPT, RPT)])
    plsc.subcore_barrier()

    def run(srcm, dstm, ng):
        @pl.loop(0, ng)
        def _(g):
            pltpu.sync_copy(srcm.at[s, pl.ds(g * GE, GE)], idxs_v)
            pltpu.sync_copy(dstm.at[s, pl.ds(g * GE, GE)], idxd_v)
            _gather2(hs_hbm, idxs_v, 0, rows0, sem0a, sem0b)

            @pl.loop(0, GE, step=2)
            def _(j0):
                for b in range(2):
                    j = j0 + b
                    cur, ca, cb = (rows0, sem0a, sem0b) if b == 0 else \
                                  (rows1, sem1a, sem1b)
                    nxt, na, nb = (rows1, sem1a, sem1b) if b == 0 else \
                                  (rows0, sem0a, sem0b)
                    _gather2_wait(hs_hbm, idxs_v, j, cur, ca, cb)
                    if b == 0:
                        _gather2(hs_hbm, idxs_v, j + 1, nxt, na, nb)
                    else:
                        @pl.when(j0 + 2 < GE)
                        def _():
                            _gather2(hs_hbm, idxs_v, j0 + 2, nxt, na, nb)
                    pltpu.sync_copy(cur, acc_sh.at[idxd_v.at[j]], add=True)

    @pl.when(c == 0)
    def _():
        run(srcA_hbm, dstA_hbm, NGA)

    @pl.when(c == 1)
    def _():
        run(srcB_hbm, dstB_hbm, NGB)

    plsc.subcore_barrier()
    pltpu.sync_copy(acc_sh.at[pl.ds(s * RPT, RPT)],
                    out_hbm.at[pl.ds(c * ACC_ROWS + s * RPT, RPT)])


@functools.lru_cache(maxsize=None)
def _make_msg_es_kernel(dh):
    mesh = plsc.VectorSubcoreMesh(core_axis_name="c", subcore_axis_name="s",
                                  num_cores=NC, num_subcores=NS)
    return pl.kernel(
        _msg_es_body,
        out_type=jax.ShapeDtypeStruct((NC * ACC_ROWS, dh), jnp.float32),
        mesh=mesh,
        scratch_types=[
            pltpu.VMEM((GE, CH), jnp.int32),
            pltpu.VMEM((GE, CH), jnp.int32),
            pltpu.VMEM((CH, dh), jnp.float32),
            pltpu.VMEM((CH, dh), jnp.float32),
            pltpu.VMEM_SHARED((ACC_ROWS, dh), jnp.float32),
            pltpu.SemaphoreType.DMA,
            pltpu.SemaphoreType.DMA,
            pltpu.SemaphoreType.DMA,
            pltpu.SemaphoreType.DMA,
        ],
    )


def _deg_body(dstd_hbm, out_hbm, idx_v, ones_v, stage_v, acc_sh, sem):
    c = lax.axis_index("c")
    s = lax.axis_index("s")
    w = c * NS + s

    pltpu.sync_copy(dstd_hbm.at[w], idx_v)
    for i in range(CH // 16):
        ones_v[pl.ds(i * 16, 16)] = jnp.full((16,), 1.0, jnp.float32)
    for i in range(DEG_SLICE // 16):
        stage_v[pl.ds(i * 16, 16)] = jnp.zeros((16,), jnp.float32)
    pltpu.sync_copy(stage_v, acc_sh.at[pl.ds(s * DEG_SLICE, DEG_SLICE)])
    plsc.subcore_barrier()

    @pl.loop(0, T_DEG)
    def _(j):
        pltpu.sync_copy(ones_v, acc_sh.at[idx_v.at[j]], add=True)

    plsc.subcore_barrier()
    pltpu.sync_copy(acc_sh.at[pl.ds(s * DEG_SLICE, DEG_SLICE)], stage_v)
    pltpu.sync_copy(stage_v,
                    out_hbm.at[pl.ds(c * DEG_ROWS + s * DEG_SLICE, DEG_SLICE)])


@functools.lru_cache(maxsize=None)
def _make_deg_kernel():
    return pl.kernel(
        _deg_body,
        out_type=jax.ShapeDtypeStruct((NC * DEG_ROWS,), jnp.float32),
        mesh=plsc.VectorSubcoreMesh(core_axis_name="c", subcore_axis_name="s",
                                    num_cores=NC, num_subcores=NS),
        scratch_types=[
            pltpu.VMEM((T_DEG, CH), jnp.int32),
            pltpu.VMEM((CH,), jnp.float32),
            pltpu.VMEM((DEG_SLICE,), jnp.float32),
            pltpu.VMEM_SHARED((DEG_ROWS,), jnp.float32),
            pltpu.SemaphoreType.DMA,
        ],
    )


def _dinv_from(degp_ref):
    deg = degp_ref[:, 0:1] + degp_ref[:, 1:2] + 1.0
    return lax.rsqrt(deg)


def _pre_body(degp_ref, x_ref, out_ref):
    dinv = _dinv_from(degp_ref)
    xs = x_ref[...] * dinv
    out_ref[0, :N] = xs
    out_ref[1, :N] = xs


_pre_kernel = pl.pallas_call(
    _pre_body,
    out_shape=jax.ShapeDtypeStruct((2, ACC_ROWS, IN_CH), jnp.float32),
)


def _l1_body(acc_ref, degp_ref, x_ref, w1_ref, b_ref, g_ref, be_ref, wn_ref,
             out_ref):
    dinv = _dinv_from(degp_ref)
    h = acc_ref[0, :N] + acc_ref[1, :N] + x_ref[...] * dinv
    t = jnp.dot(h * dinv, w1_ref[...],
                preferred_element_type=jnp.float32) + b_ref[...]
    m = jnp.mean(t, axis=0, keepdims=True)
    v = jnp.mean((t - m) * (t - m), axis=0, keepdims=True)
    t = (t - m) * lax.rsqrt(v + EPS) * g_ref[...] + be_ref[...]
    t = jnp.maximum(t, 0.0)
    hs = jnp.dot(t, wn_ref[...], preferred_element_type=jnp.float32) * dinv
    out_ref[0, :N] = hs[:, :HID // 2]
    out_ref[1, :N] = hs[:, HID // 2:]


_l1_kernel = pl.pallas_call(
    _l1_body,
    out_shape=jax.ShapeDtypeStruct((2, ACC_ROWS, HID // 2), jnp.float32),
)


def _mid_body(din, dout, split, acc_ref, degp_ref, hs_ref, b_ref, g_ref,
              be_ref, wn_ref, out_ref):
    dinv = _dinv_from(degp_ref)
    h = jnp.concatenate([acc_ref[0, :N] + hs_ref[0, :N],
                         acc_ref[1, :N] + hs_ref[1, :N]], axis=1)
    t = h * dinv + b_ref[...]
    m = jnp.mean(t, axis=0, keepdims=True)
    v = jnp.mean((t - m) * (t - m), axis=0, keepdims=True)
    t = (t - m) * lax.rsqrt(v + EPS) * g_ref[...] + be_ref[...]
    t = jnp.maximum(t, 0.0)
    hs = jnp.dot(t, wn_ref[...], preferred_element_type=jnp.float32) * dinv
    if split:
        out_ref[0, :N] = hs[:, :dout // 2]
        out_ref[1, :N] = hs[:, dout // 2:]
    else:
        out_ref[0, :N] = hs
        out_ref[1, :N] = hs


def _make_mid_kernel(din, dout, split=True):
    shape = (2, ACC_ROWS, dout // 2) if split else (2, ACC_ROWS, dout)
    return pl.pallas_call(
        functools.partial(_mid_body, din, dout, split),
        out_shape=jax.ShapeDtypeStruct(shape, jnp.float32),
    )


def _post_body(acc_ref, degp_ref, hs_ref, b_ref, g_ref, be_ref, batch_ref,
               wl_ref, bl_ref, out_ref):
    dinv = _dinv_from(degp_ref)
    h = acc_ref[0, :N] + acc_ref[1, :N] + hs_ref[0, :N]
    t = h * dinv + b_ref[...]
    m = jnp.mean(t, axis=0, keepdims=True)
    v = jnp.mean((t - m) * (t - m), axis=0, keepdims=True)
    t = (t - m) * lax.rsqrt(v + EPS) * g_ref[...] + be_ref[...]
    t = jnp.maximum(t, 0.0)
    gids = lax.broadcasted_iota(jnp.int32, (N_GRAPHS, N), 0)
    onehot = jnp.where(batch_ref[...] == gids, 1.0, 0.0)
    sums = jnp.dot(onehot, t, preferred_element_type=jnp.float32)
    cnt = jnp.sum(onehot, axis=1, keepdims=True)
    pooled = sums / jnp.maximum(cnt, 1.0)
    out_ref[...] = jnp.dot(pooled, wl_ref[...],
                           preferred_element_type=jnp.float32) + bl_ref[...]


_post_kernel = pl.pallas_call(
    _post_body,
    out_shape=jax.ShapeDtypeStruct((N_GRAPHS, N_CLASSES), jnp.float32),
)


def kernel(x, edge_index, batch, W1, b1, g1, be1, W2, b2, g2, be2,
           W3, b3, g3, be3, Wl, bl):
    src = edge_index[0].astype(jnp.int32)
    dst = edge_index[1].astype(jnp.int32)

    srcp = jnp.concatenate([src, jnp.zeros((EP_MSG - E,), jnp.int32)])
    dstp = jnp.concatenate([dst, jnp.full((EP_MSG - E,), N, jnp.int32)])
    srcm = (jnp.stack([srcp, srcp + ACC_ROWS])
            .reshape(NC * NS, T_MSG, CH))
    dstm = dstp.reshape(NS, T_MSG, CH)
    chunks_s = srcp.reshape(EP_MSG // CH, CH)
    chunks_d = dstp.reshape(EP_MSG // CH, CH)
    srcA = chunks_s[:NS * CPA].reshape(NS, CPA, CH)
    dstA = chunks_d[:NS * CPA].reshape(NS, CPA, CH)
    srcB = chunks_s[NS * CPA:].reshape(NS, CPB, CH) + ACC_ROWS
    dstB = chunks_d[NS * CPA:].reshape(NS, CPB, CH)
    dstd = (jnp.concatenate([dst, jnp.full((EP_DEG - E,), N, jnp.int32)])
            .reshape(NC * NS, T_DEG, CH))
    zeros_rpt = jnp.zeros((RPT, HID // 2), jnp.float32)
    degp = _make_deg_kernel()(dstd)
    degp2 = degp.reshape(NC, DEG_ROWS)[:, :N].T

    xs = _pre_kernel(degp2, x).reshape(NC * ACC_ROWS, IN_CH)
    accx = (_make_msg_es_kernel(IN_CH)(xs, srcA, dstA, srcB, dstB, zeros_rpt)
            .reshape(NC, ACC_ROWS, IN_CH))
    hs2 = _l1_kernel(accx, degp2, x, W1, b1, g1, be1, W2)
    acc2 = (_make_msg_kernel(HID // 2)(hs2.reshape(NC * ACC_ROWS, HID // 2),
                                       srcm, dstm, zeros_rpt)
            .reshape(NC, ACC_ROWS, HID // 2))
    hs3 = _make_mid_kernel(HID, OUT_CH, split=False)(acc2, degp2, hs2,
                                                     b2, g2, be2, W3)
    acc3 = (_make_msg_es_kernel(OUT_CH)(hs3.reshape(NC * ACC_ROWS, OUT_CH),
                                        srcA, dstA, srcB, dstB, zeros_rpt)
            .reshape(NC, ACC_ROWS, OUT_CH))
    return _post_kernel(acc3, degp2, hs3, b3, g3, be3,
                        batch.astype(jnp.int32).reshape(1, N), Wl, bl)


# speedup vs baseline: 1.6586x; 1.0999x over previous
import functools

import jax
import jax.numpy as jnp
from jax import lax
from jax.experimental import pallas as pl
from jax.experimental.pallas import tpu as pltpu
from jax.experimental.pallas import tpu_sc as plsc

N = 10000
E = 320000
IN_CH = 128
HID = 256
OUT_CH = 128
N_CLASSES = 16
N_GRAPHS = 64
EPS = 1e-5

NC = 2
NS = 16
CH = 128

G = 16
NG = -(-((E + NS * CH - 1) // (NS * CH)) // G)
T_MSG = NG * G
EP_MSG = NS * CH * T_MSG
T_DEG = (E + NC * NS * CH - 1) // (NC * NS * CH)
EP_DEG = NC * NS * CH * T_DEG

ACC_ROWS = 10112
RPT = ACC_ROWS // NS
GE = 8
CPA = 128
CPB = 32
NGA = CPA // GE
NGB = CPB // GE
DEG_ROWS = 10240
DEG_SLICE = DEG_ROWS // NS


def _gather2(hs_hbm, idxs_v, j, buf, semA, semB):
    del semB
    pltpu.async_copy(hs_hbm.at[idxs_v.at[j]], buf, semA)


def _gather2_wait(hs_hbm, idxs_v, j, buf, semA, semB):
    del semB
    pltpu.make_async_copy(hs_hbm.at[idxs_v.at[j]], buf, semA).wait()


def _msg_body(dh, hs_hbm, srcm_hbm, dstm_hbm, zeros_hbm, out_hbm,
              idxs_v, idxd_v, rows0, rows1, acc_sh,
              sem0a, sem0b, sem1a, sem1b):
    c = lax.axis_index("c")
    s = lax.axis_index("s")
    w = c * NS + s

    pltpu.sync_copy(zeros_hbm, acc_sh.at[pl.ds(s * RPT, RPT)])
    plsc.subcore_barrier()

    @pl.loop(0, NG)
    def _(g):
        pltpu.sync_copy(srcm_hbm.at[w, pl.ds(g * G, G)], idxs_v)
        pltpu.sync_copy(dstm_hbm.at[s, pl.ds(g * G, G)], idxd_v)
        _gather2(hs_hbm, idxs_v, 0, rows0, sem0a, sem0b)

        @pl.loop(0, G, step=2)
        def _(j0):
            for b in range(2):
                j = j0 + b
                cur, ca, cb = (rows0, sem0a, sem0b) if b == 0 else \
                              (rows1, sem1a, sem1b)
                nxt, na, nb = (rows1, sem1a, sem1b) if b == 0 else \
                              (rows0, sem0a, sem0b)
                _gather2_wait(hs_hbm, idxs_v, j, cur, ca, cb)
                if b == 0:
                    _gather2(hs_hbm, idxs_v, j + 1, nxt, na, nb)
                else:
                    @pl.when(j0 + 2 < G)
                    def _():
                        _gather2(hs_hbm, idxs_v, j0 + 2, nxt, na, nb)
                pltpu.sync_copy(cur, acc_sh.at[idxd_v.at[j]], add=True)

    plsc.subcore_barrier()
    pltpu.sync_copy(acc_sh.at[pl.ds(s * RPT, RPT)],
                    out_hbm.at[pl.ds(c * ACC_ROWS + s * RPT, RPT)])


@functools.lru_cache(maxsize=None)
def _make_msg_kernel(dh):
    mesh = plsc.VectorSubcoreMesh(core_axis_name="c", subcore_axis_name="s",
                                  num_cores=NC, num_subcores=NS)
    return pl.kernel(
        functools.partial(_msg_body, dh),
        out_type=jax.ShapeDtypeStruct((NC * ACC_ROWS, dh), jnp.float32),
        mesh=mesh,
        scratch_types=[
            pltpu.VMEM((G, CH), jnp.int32),
            pltpu.VMEM((G, CH), jnp.int32),
            pltpu.VMEM((CH, dh), jnp.float32),
            pltpu.VMEM((CH, dh), jnp.float32),
            pltpu.VMEM_SHARED((ACC_ROWS, dh), jnp.float32),
            pltpu.SemaphoreType.DMA,
            pltpu.SemaphoreType.DMA,
            pltpu.SemaphoreType.DMA,
            pltpu.SemaphoreType.DMA,
        ],
    )


def _msg_es_body(hs_hbm, srcA_hbm, dstA_hbm, srcB_hbm, dstB_hbm, zeros_hbm,
                 out_hbm, idxs_v, idxd_v, rows0, rows1, acc_sh,
                 sem0a, sem0b, sem1a, sem1b):
    c = lax.axis_index("c")
    s = lax.axis_index("s")

    pltpu.sync_copy(zeros_hbm, acc_sh.at[pl.ds(s * RPT, RPT)])
    plsc.subcore_barrier()

    def run(srcm, dstm, ng):
        @pl.loop(0, ng)
        def _(g):
            pltpu.sync_copy(srcm.at[s, pl.ds(g * GE, GE)], idxs_v)
            pltpu.sync_copy(dstm.at[s, pl.ds(g * GE, GE)], idxd_v)
            _gather2(hs_hbm, idxs_v, 0, rows0, sem0a, sem0b)

            @pl.loop(0, GE, step=2)
            def _(j0):
                for b in range(2):
                    j = j0 + b
                    cur, ca, cb = (rows0, sem0a, sem0b) if b == 0 else \
                                  (rows1, sem1a, sem1b)
                    nxt, na, nb = (rows1, sem1a, sem1b) if b == 0 else \
                                  (rows0, sem0a, sem0b)
                    _gather2_wait(hs_hbm, idxs_v, j, cur, ca, cb)
                    if b == 0:
                        _gather2(hs_hbm, idxs_v, j + 1, nxt, na, nb)
                    else:
                        @pl.when(j0 + 2 < GE)
                        def _():
                            _gather2(hs_hbm, idxs_v, j0 + 2, nxt, na, nb)
                    pltpu.sync_copy(cur, acc_sh.at[idxd_v.at[j]], add=True)

    @pl.when(c == 0)
    def _():
        run(srcA_hbm, dstA_hbm, NGA)

    @pl.when(c == 1)
    def _():
        run(srcB_hbm, dstB_hbm, NGB)

    plsc.subcore_barrier()
    pltpu.sync_copy(acc_sh.at[pl.ds(s * RPT, RPT)],
                    out_hbm.at[pl.ds(c * ACC_ROWS + s * RPT, RPT)])


@functools.lru_cache(maxsize=None)
def _make_msg_es_kernel(dh):
    mesh = plsc.VectorSubcoreMesh(core_axis_name="c", subcore_axis_name="s",
                                  num_cores=NC, num_subcores=NS)
    return pl.kernel(
        _msg_es_body,
        out_type=jax.ShapeDtypeStruct((NC * ACC_ROWS, dh), jnp.float32),
        mesh=mesh,
        scratch_types=[
            pltpu.VMEM((GE, CH), jnp.int32),
            pltpu.VMEM((GE, CH), jnp.int32),
            pltpu.VMEM((CH, dh), jnp.float32),
            pltpu.VMEM((CH, dh), jnp.float32),
            pltpu.VMEM_SHARED((ACC_ROWS, dh), jnp.float32),
            pltpu.SemaphoreType.DMA,
            pltpu.SemaphoreType.DMA,
            pltpu.SemaphoreType.DMA,
            pltpu.SemaphoreType.DMA,
        ],
    )


def _deg_body(dstd_hbm, out_hbm, idx_v, ones_v, stage_v, acc_sh, sem):
    c = lax.axis_index("c")
    s = lax.axis_index("s")
    w = c * NS + s

    pltpu.sync_copy(dstd_hbm.at[w], idx_v)
    for i in range(CH // 16):
        ones_v[pl.ds(i * 16, 16)] = jnp.full((16,), 1.0, jnp.float32)
    for i in range(DEG_SLICE // 16):
        stage_v[pl.ds(i * 16, 16)] = jnp.zeros((16,), jnp.float32)
    pltpu.sync_copy(stage_v, acc_sh.at[pl.ds(s * DEG_SLICE, DEG_SLICE)])
    plsc.subcore_barrier()

    @pl.loop(0, T_DEG)
    def _(j):
        pltpu.sync_copy(ones_v, acc_sh.at[idx_v.at[j]], add=True)

    plsc.subcore_barrier()
    pltpu.sync_copy(acc_sh.at[pl.ds(s * DEG_SLICE, DEG_SLICE)], stage_v)
    pltpu.sync_copy(stage_v,
                    out_hbm.at[pl.ds(c * DEG_ROWS + s * DEG_SLICE, DEG_SLICE)])


@functools.lru_cache(maxsize=None)
def _make_deg_kernel():
    return pl.kernel(
        _deg_body,
        out_type=jax.ShapeDtypeStruct((NC * DEG_ROWS,), jnp.float32),
        mesh=plsc.VectorSubcoreMesh(core_axis_name="c", subcore_axis_name="s",
                                    num_cores=NC, num_subcores=NS),
        scratch_types=[
            pltpu.VMEM((T_DEG, CH), jnp.int32),
            pltpu.VMEM((CH,), jnp.float32),
            pltpu.VMEM((DEG_SLICE,), jnp.float32),
            pltpu.VMEM_SHARED((DEG_ROWS,), jnp.float32),
            pltpu.SemaphoreType.DMA,
        ],
    )


def _dinv_from(degp_ref):
    deg = degp_ref[:, 0:1] + degp_ref[:, 1:2] + 1.0
    return lax.rsqrt(deg)


def _pre_body(degp_ref, x_ref, out_ref):
    dinv = _dinv_from(degp_ref)
    xs = x_ref[...] * dinv
    out_ref[0, :N] = xs
    out_ref[1, :N] = xs


_pre_kernel = pl.pallas_call(
    _pre_body,
    out_shape=jax.ShapeDtypeStruct((2, ACC_ROWS, IN_CH), jnp.float32),
)


def _l1_body(acc_ref, degp_ref, x_ref, w1_ref, b_ref, g_ref, be_ref, wn_ref,
             out_ref):
    dinv = _dinv_from(degp_ref)
    h = acc_ref[0, :N] + acc_ref[1, :N] + x_ref[...] * dinv
    t = jnp.dot(h * dinv, w1_ref[...],
                preferred_element_type=jnp.float32) + b_ref[...]
    m = jnp.mean(t, axis=0, keepdims=True)
    v = jnp.mean((t - m) * (t - m), axis=0, keepdims=True)
    t = (t - m) * lax.rsqrt(v + EPS) * g_ref[...] + be_ref[...]
    t = jnp.maximum(t, 0.0)
    hs = jnp.dot(t, wn_ref[...], preferred_element_type=jnp.float32) * dinv
    out_ref[0, :N] = hs[:, :HID // 2]
    out_ref[1, :N] = hs[:, HID // 2:]


_l1_kernel = pl.pallas_call(
    _l1_body,
    out_shape=jax.ShapeDtypeStruct((2, ACC_ROWS, HID // 2), jnp.float32),
)


def _mid_body(din, dout, split, acc_ref, degp_ref, hs_ref, b_ref, g_ref,
              be_ref, wn_ref, out_ref):
    dinv = _dinv_from(degp_ref)
    h = jnp.concatenate([acc_ref[0, :N] + hs_ref[0, :N],
                         acc_ref[1, :N] + hs_ref[1, :N]], axis=1)
    t = h * dinv + b_ref[...]
    m = jnp.mean(t, axis=0, keepdims=True)
    v = jnp.mean((t - m) * (t - m), axis=0, keepdims=True)
    t = (t - m) * lax.rsqrt(v + EPS) * g_ref[...] + be_ref[...]
    t = jnp.maximum(t, 0.0)
    hs = jnp.dot(t, wn_ref[...], preferred_element_type=jnp.float32) * dinv
    if split:
        out_ref[0, :N] = hs[:, :dout // 2]
        out_ref[1, :N] = hs[:, dout // 2:]
    else:
        out_ref[0, :N] = hs
        out_ref[1, :N] = hs


def _make_mid_kernel(din, dout, split=True):
    shape = (2, ACC_ROWS, dout // 2) if split else (2, ACC_ROWS, dout)
    return pl.pallas_call(
        functools.partial(_mid_body, din, dout, split),
        out_shape=jax.ShapeDtypeStruct(shape, jnp.float32),
    )


def _post_body(acc_ref, degp_ref, hs_ref, b_ref, g_ref, be_ref, batch_ref,
               wl_ref, bl_ref, out_ref):
    dinv = _dinv_from(degp_ref)
    h = acc_ref[0, :N] + acc_ref[1, :N] + hs_ref[0, :N]
    t = h * dinv + b_ref[...]
    m = jnp.mean(t, axis=0, keepdims=True)
    v = jnp.mean((t - m) * (t - m), axis=0, keepdims=True)
    t = (t - m) * lax.rsqrt(v + EPS) * g_ref[...] + be_ref[...]
    t = jnp.maximum(t, 0.0)
    gids = lax.broadcasted_iota(jnp.int32, (N_GRAPHS, N), 0)
    onehot = jnp.where(batch_ref[...] == gids, 1.0, 0.0)
    sums = jnp.dot(onehot, t, preferred_element_type=jnp.float32)
    cnt = jnp.sum(onehot, axis=1, keepdims=True)
    pooled = sums / jnp.maximum(cnt, 1.0)
    out_ref[...] = jnp.dot(pooled, wl_ref[...],
                           preferred_element_type=jnp.float32) + bl_ref[...]


_post_kernel = pl.pallas_call(
    _post_body,
    out_shape=jax.ShapeDtypeStruct((N_GRAPHS, N_CLASSES), jnp.float32),
)


def kernel(x, edge_index, batch, W1, b1, g1, be1, W2, b2, g2, be2,
           W3, b3, g3, be3, Wl, bl):
    src = edge_index[0].astype(jnp.int32)
    dst = edge_index[1].astype(jnp.int32)

    srcp = jnp.concatenate([src, jnp.zeros((EP_MSG - E,), jnp.int32)])
    dstp = jnp.concatenate([dst, jnp.full((EP_MSG - E,), N, jnp.int32)])
    srcm = (jnp.stack([srcp, srcp + ACC_ROWS])
            .reshape(NC * NS, T_MSG, CH))
    dstm = dstp.reshape(NS, T_MSG, CH)
    chunks_s = srcp.reshape(EP_MSG // CH, CH)
    chunks_d = dstp.reshape(EP_MSG // CH, CH)
    srcA = chunks_s[:NS * CPA].reshape(NS, CPA, CH)
    dstA = chunks_d[:NS * CPA].reshape(NS, CPA, CH)
    srcB = chunks_s[NS * CPA:].reshape(NS, CPB, CH) + ACC_ROWS
    dstB = chunks_d[NS * CPA:].reshape(NS, CPB, CH)
    dstd = (jnp.concatenate([dst, jnp.full((EP_DEG - E,), N, jnp.int32)])
            .reshape(NC * NS, T_DEG, CH))
    zeros_rpt = jnp.zeros((RPT, HID // 2), jnp.float32)
    degp = _make_deg_kernel()(dstd)
    degp2 = degp.reshape(NC, DEG_ROWS)[:, :N].T

    xs = _pre_kernel(degp2, x).reshape(NC * ACC_ROWS, IN_CH)
    accx = (_make_msg_es_kernel(IN_CH)(xs, srcA, dstA, srcB, dstB, zeros_rpt)
            .reshape(NC, ACC_ROWS, IN_CH))
    hs2 = _l1_kernel(accx, degp2, x, W1, b1, g1, be1, W2)
    acc2 = (_make_msg_kernel(HID // 2)(hs2.reshape(NC * ACC_ROWS, HID // 2),
                                       srcm, dstm, zeros_rpt)
            .reshape(NC, ACC_ROWS, HID // 2))
    hs3 = _make_mid_kernel(HID, OUT_CH, split=False)(acc2, degp2, hs2,
                                                     b2, g2, be2, W3)
    acc3 = (_make_msg_es_kernel(OUT_CH)(hs3.reshape(NC * ACC_ROWS, OUT_CH),
                                        srcA, dstA, srcB, dstB, zeros_rpt)
            .reshape(NC, ACC_ROWS, OUT_CH))
    return _post_kernel(acc3, degp2, hs3, b3, g3, be3,
                        batch.astype(jnp.int32).reshape(1, N), Wl, bl)
